# baseline (speedup 1.0000x reference)
import functools
import math

import jax
import jax.numpy as jnp
from jax import lax
from jax.experimental import pallas as pl
from jax.experimental.pallas import tpu as pltpu
from jax.experimental.pallas import tpu_sc as plsc

N = 10000
E = 320000
H = 128
NLAYERS = 3
NW = 16
EPW = E // NW
CHUNK = 40
NCHUNKS = EPW // CHUNK
GCH = 50
NGROUPS = NCHUNKS // GCH
PAIRS = GCH // 2
_SEL_A = [32 * k + i for k in range(4) for i in range(16)]
_SEL_B = [32 * k + 16 + i for k in range(4) for i in range(16)]
RPT = N // 16
NB = 1000
EB = 4000

_BN_SCALE = 1.0 / math.sqrt(1.0 + 1e-5)


def _node_enc_body(x_ref, w_ref, b_ref, o_ref):
    o_ref[...] = jnp.maximum(
        jnp.dot(x_ref[...], w_ref[...], preferred_element_type=jnp.float32)
        + b_ref[...], 0.0)


def _node_enc(x, w, b):
    return pl.pallas_call(
        _node_enc_body,
        grid=(N // NB,),
        in_specs=[
            pl.BlockSpec((NB, H), lambda i: (i, 0)),
            pl.BlockSpec((H, H), lambda i: (0, 0)),
            pl.BlockSpec((1, H), lambda i: (0, 0)),
        ],
        out_specs=pl.BlockSpec((NB, H), lambda i: (i, 0)),
        out_shape=jax.ShapeDtypeStruct((N, H), jnp.float32),
    )(x, w, b)


def _round_bf16_bits(x):
    xi = jax.lax.bitcast_convert_type(x, jnp.int32)
    lsb = jnp.bitwise_and(jax.lax.shift_right_logical(xi, 16), 1)
    return jax.lax.shift_right_logical(xi + jnp.int32(0x7FFF) + lsb, 16)


def _edge_e_body(ea_ref, wee_ref, wea_ref, web_ref, bee_ref, bea_ref, beb_ref,
                 e0_ref, e1_ref, e2_ref):
    ea = ea_ref[...]
    outs = (e0_ref, e1_ref, e2_ref)
    for l in range(NLAYERS):
        halves = []
        for w_ref, b_ref in ((wea_ref, bea_ref), (web_ref, beb_ref)):
            wl = w_ref[l]
            wp = jnp.dot(wee_ref[...], wl, preferred_element_type=jnp.float32)
            bp = (jnp.dot(bee_ref[...], wl, preferred_element_type=jnp.float32)
                  + b_ref[l][None, :])
            halves.append(
                jnp.dot(ea, wp, preferred_element_type=jnp.float32) + bp)
        outs[l][...] = jnp.bitwise_or(
            _round_bf16_bits(halves[0]),
            jax.lax.shift_left(_round_bf16_bits(halves[1]), 16))


def _edge_e(edge_attr, w_ee, we_a, we_b, b_ee, be_a, be_b):
    d_edge = edge_attr.shape[1]
    espec = pl.BlockSpec((EB, H // 2), lambda i: (i, 0))
    wspec = pl.BlockSpec((NLAYERS, H, H // 2), lambda i: (0, 0, 0))
    bspec = pl.BlockSpec((NLAYERS, H // 2), lambda i: (0, 0))
    return pl.pallas_call(
        _edge_e_body,
        grid=(E // EB,),
        in_specs=[
            pl.BlockSpec((EB, d_edge), lambda i: (i, 0)),
            pl.BlockSpec((d_edge, H), lambda i: (0, 0)),
            wspec, wspec,
            pl.BlockSpec((1, H), lambda i: (0, 0)),
            bspec, bspec,
        ],
        out_specs=[espec, espec, espec],
        out_shape=[jax.ShapeDtypeStruct((E, H // 2), jnp.int32)] * NLAYERS,
    )(edge_attr, w_ee, we_a, we_b, b_ee, be_a, be_b)


@functools.cache
def _get_sc_msgpass():
    mesh = plsc.VectorSubcoreMesh(core_axis_name="c", subcore_axis_name="s",
                                  num_cores=1)
    return functools.partial(
        pl.kernel,
        out_type=jax.ShapeDtypeStruct((16, RPT, H), jnp.float32),
        mesh=mesh,
        compiler_params=pltpu.CompilerParams(needs_layout_passes=False),
        scratch_types=[
            pltpu.VMEM((GCH, CHUNK), jnp.int32),
            pltpu.VMEM((GCH, CHUNK), jnp.int32),
            pltpu.VMEM((CHUNK, H), jnp.float32),
            pltpu.VMEM((CHUNK, H), jnp.float32),
            pltpu.VMEM((CHUNK, H // 2), jnp.int32),
            pltpu.VMEM((CHUNK, H // 2), jnp.int32),
            pltpu.VMEM_SHARED((N, H), jnp.float32),
        ] + [pltpu.SemaphoreType.DMA] * 6,
    )(_sc_msgpass_body)


def _sc_msgpass_body(h_hbm, src_hbm, dst_hbm, e_hbm, out_hbm, srcs_v, dsts_v,
                     mbuf0, mbuf1, ebuf0, ebuf1, aggr_sh,
                     es0, es1, gs0, gs1, ss0, ss1):
    sid = lax.axis_index("s")
    wid = sid
    mbufs = (mbuf0, mbuf1)
    ebufs = (ebuf0, ebuf1)
    esems = (es0, es1)
    gsems = (gs0, gs1)
    ssems = (ss0, ss1)

    def _zrow(r, carry):
        for j in range(H // 16):
            mbuf0[r, pl.ds(j * 16, 16)] = jnp.zeros((16,), jnp.float32)
        return carry
    lax.fori_loop(0, CHUNK, _zrow, 0)

    def _zcopy(k, carry):
        pltpu.sync_copy(mbuf0,
                        aggr_sh.at[pl.ds(sid * RPT + k * CHUNK, CHUNK), :])
        return carry
    lax.fori_loop(0, RPT // CHUNK, _zcopy, 0)
    _ztail = RPT - (RPT // CHUNK) * CHUNK
    pltpu.sync_copy(
        mbuf0.at[pl.ds(0, _ztail), :],
        aggr_sh.at[pl.ds(sid * RPT + (RPT // CHUNK) * CHUNK, _ztail), :])
    plsc.subcore_barrier()

    ebase = wid * EPW
    _MASK = jnp.full((16,), -65536, jnp.int32)
    _SH16 = jnp.full((16,), 16, jnp.int32)

    def _compute(b):
        def _rrow(r, rc):
            for j in range(H // 32):
                iv = ebufs[b][r, pl.ds(j * 16, 16)]
                lo = plsc.bitcast(jax.lax.shift_left(iv, _SH16), jnp.float32)
                hi = plsc.bitcast(jnp.bitwise_and(iv, _MASK), jnp.float32)
                sl0 = pl.ds(j * 32, 16)
                sl1 = pl.ds(j * 32 + 16, 16)
                mbufs[b][r, sl0] = jnp.maximum(mbufs[b][r, sl0] + lo, 0.0)
                mbufs[b][r, sl1] = jnp.maximum(mbufs[b][r, sl1] + hi, 0.0)
            return rc
        lax.fori_loop(0, CHUNK, _rrow, 0)

    def _group(gg, gcarry):
        pltpu.sync_copy(src_hbm.at[wid, gg], srcs_v)
        pltpu.sync_copy(dst_hbm.at[wid, gg], dsts_v)
        gbase = ebase + gg * (GCH * CHUNK)

        def e_start(b, l):
            pltpu.async_copy(e_hbm.at[pl.ds(gbase + l * CHUNK, CHUNK), :],
                             ebufs[b], esems[b])

        def e_wait(b, l):
            pltpu.make_async_copy(
                e_hbm.at[pl.ds(gbase + l * CHUNK, CHUNK), :],
                ebufs[b], esems[b]).wait()

        def g_start(b, l):
            pltpu.async_copy(h_hbm.at[srcs_v.at[l]], mbufs[b], gsems[b])

        def g_wait(b, l):
            pltpu.make_async_copy(h_hbm.at[srcs_v.at[l]], mbufs[b],
                                  gsems[b]).wait()

        def s_start(b, l):
            pltpu.async_copy(mbufs[b], aggr_sh.at[dsts_v.at[l]], ssems[b],
                             add=True)

        def s_wait(b, l):
            pltpu.make_async_copy(mbufs[b], aggr_sh.at[dsts_v.at[l]],
                                  ssems[b]).wait()

        e_start(0, 0)
        g_start(0, 0)

        def _pair(g, pcarry):
            l0 = g * 2
            @pl.when(g >= 1)
            def _():
                s_wait(1, l0 - 1)
            e_start(1, l0 + 1)
            g_start(1, l0 + 1)
            e_wait(0, l0)
            g_wait(0, l0)
            _compute(0)
            s_start(0, l0)
            @pl.when(g < PAIRS - 1)
            def _():
                s_wait(0, l0)
                e_start(0, l0 + 2)
                g_start(0, l0 + 2)
            e_wait(1, l0 + 1)
            g_wait(1, l0 + 1)
            _compute(1)
            s_start(1, l0 + 1)
            return pcarry

        lax.fori_loop(0, PAIRS, _pair, 0)
        s_wait(0, GCH - 2)
        s_wait(1, GCH - 1)
        return gcarry

    lax.fori_loop(0, NGROUPS, _group, 0)
    plsc.subcore_barrier()

    pltpu.sync_copy(aggr_sh.at[pl.ds(sid * RPT, RPT), :], out_hbm.at[sid])


def _layer_body(h_ref, p_ref, eps_ref, wm1_ref, bm1_ref, wm2_ref, bm2_ref,
                g_ref, b_ref, o_ref):
    h = h_ref[...]
    out = (1.0 + eps_ref[0, 0]) * h + p_ref[...]
    t = jnp.maximum(
        jnp.dot(out, wm1_ref[...], preferred_element_type=jnp.float32)
        + bm1_ref[...], 0.0)
    out = (jnp.dot(t, wm2_ref[...], preferred_element_type=jnp.float32)
           + bm2_ref[...])
    out = out * (g_ref[...] * _BN_SCALE) + b_ref[...] + h
    o_ref[...] = jnp.maximum(out, 0.0)


def _layer_tc(h, parts, eps_l, wm1, bm1, wm2, bm2, gamma_l, beta_l):
    return pl.pallas_call(
        _layer_body,
        grid=(N // NB,),
        in_specs=[
            pl.BlockSpec((NB, H), lambda i: (i, 0)),
            pl.BlockSpec((NB, H), lambda i: (i, 0)),
            pl.BlockSpec(memory_space=pltpu.SMEM),
            pl.BlockSpec((H, 2 * H), lambda i: (0, 0)),
            pl.BlockSpec((1, 2 * H), lambda i: (0, 0)),
            pl.BlockSpec((2 * H, H), lambda i: (0, 0)),
            pl.BlockSpec((1, H), lambda i: (0, 0)),
            pl.BlockSpec((1, H), lambda i: (0, 0)),
            pl.BlockSpec((1, H), lambda i: (0, 0)),
        ],
        out_specs=pl.BlockSpec((NB, H), lambda i: (i, 0)),
        out_shape=jax.ShapeDtypeStruct((N, H), jnp.float32),
    )(h, parts, eps_l, wm1, bm1, wm2, bm2, gamma_l, beta_l)


def _pool_body(h_ref, wo1_ref, bo1_ref, wo2_ref, bo2_ref, logits_ref,
               pooled_ref, acc_ref):
    i = pl.program_id(0)

    @pl.when(i == 0)
    def _():
        acc_ref[...] = jnp.zeros_like(acc_ref)

    acc_ref[...] += jnp.sum(h_ref[...], axis=0, keepdims=True)

    @pl.when(i == pl.num_programs(0) - 1)
    def _():
        pooled = acc_ref[...] * (1.0 / N)
        pooled_ref[...] = pooled
        t = jnp.maximum(
            jnp.dot(pooled, wo1_ref[...], preferred_element_type=jnp.float32)
            + bo1_ref[...], 0.0)
        logits_ref[...] = (
            jnp.dot(t, wo2_ref[...], preferred_element_type=jnp.float32)
            + bo2_ref[...])


def _pool_tc(h, wo1, bo1, wo2, bo2):
    h2 = wo1.shape[1]
    nout = wo2.shape[1]
    return pl.pallas_call(
        _pool_body,
        grid=(N // NB,),
        in_specs=[
            pl.BlockSpec((NB, H), lambda i: (i, 0)),
            pl.BlockSpec((H, h2), lambda i: (0, 0)),
            pl.BlockSpec((1, h2), lambda i: (0, 0)),
            pl.BlockSpec((h2, nout), lambda i: (0, 0)),
            pl.BlockSpec((1, nout), lambda i: (0, 0)),
        ],
        out_specs=[
            pl.BlockSpec((1, nout), lambda i: (0, 0)),
            pl.BlockSpec((1, H), lambda i: (0, 0)),
        ],
        out_shape=[
            jax.ShapeDtypeStruct((1, nout), jnp.float32),
            jax.ShapeDtypeStruct((1, H), jnp.float32),
        ],
        scratch_shapes=[pltpu.VMEM((1, H), jnp.float32)],
    )(h, wo1, bo1, wo2, bo2)


def kernel(x, edge_index, edge_attr, W_ne, b_ne, W_ee, b_ee, eps, We, be,
           Wm1, bm1, Wm2, bm2, gamma, beta, Wo1, bo1, Wo2, bo2):
    ei = edge_index.astype(jnp.int32)
    src_r = ei[0].reshape(NW, NGROUPS, GCH, CHUNK)
    dst_r = ei[1].reshape(NW, NGROUPS, GCH, CHUNK)

    h = _node_enc(x, W_ne, b_ne.reshape(1, H))
    sa = jnp.array(_SEL_A, dtype=jnp.int32)
    sb = jnp.array(_SEL_B, dtype=jnp.int32)
    e_all = _edge_e(edge_attr, W_ee, We[:, :, sa], We[:, :, sb],
                    b_ee.reshape(1, H), be[:, sa], be[:, sb])

    sc_msgpass = _get_sc_msgpass()
    for l in range(NLAYERS):
        parts = sc_msgpass(h, src_r, dst_r, e_all[l]).reshape(N, H)
        h = _layer_tc(h, parts, eps[l].reshape(1, 1), Wm1[l],
                      bm1[l].reshape(1, 2 * H), Wm2[l], bm2[l].reshape(1, H),
                      gamma[l].reshape(1, H), beta[l].reshape(1, H))

    return _pool_tc(h, Wo1, bo1.reshape(1, H // 2), Wo2,
                    bo2.reshape(1, Wo2.shape[1]))


# speedup vs baseline: 1.0998x; 1.0998x over previous
import functools
import math

import jax
import jax.numpy as jnp
from jax import lax
from jax.experimental import pallas as pl
from jax.experimental.pallas import tpu as pltpu
from jax.experimental.pallas import tpu_sc as plsc

N = 10000
E = 320000
H = 128
NLAYERS = 3
NW = 16
EPW = E // NW
CHUNK = 40
NCHUNKS = EPW // CHUNK
GCH = 20
NGROUPS = NCHUNKS // GCH
QUADS = GCH // 4
_SEL_A = [32 * k + i for k in range(4) for i in range(16)]
_SEL_B = [32 * k + 16 + i for k in range(4) for i in range(16)]
RPT = N // 16
NB = 1000
EB = 4000

_BN_SCALE = 1.0 / math.sqrt(1.0 + 1e-5)


def _node_enc_body(x_ref, w_ref, b_ref, o_ref):
    o_ref[...] = jnp.maximum(
        jnp.dot(x_ref[...], w_ref[...], preferred_element_type=jnp.float32)
        + b_ref[...], 0.0)


def _node_enc(x, w, b):
    return pl.pallas_call(
        _node_enc_body,
        grid=(N // NB,),
        in_specs=[
            pl.BlockSpec((NB, H), lambda i: (i, 0)),
            pl.BlockSpec((H, H), lambda i: (0, 0)),
            pl.BlockSpec((1, H), lambda i: (0, 0)),
        ],
        out_specs=pl.BlockSpec((NB, H), lambda i: (i, 0)),
        out_shape=jax.ShapeDtypeStruct((N, H), jnp.float32),
    )(x, w, b)


def _round_bf16_bits(x):
    xi = jax.lax.bitcast_convert_type(x, jnp.int32)
    lsb = jnp.bitwise_and(jax.lax.shift_right_logical(xi, 16), 1)
    return jax.lax.shift_right_logical(xi + jnp.int32(0x7FFF) + lsb, 16)


def _edge_e_body(ea_ref, wee_ref, wea_ref, web_ref, bee_ref, bea_ref, beb_ref,
                 e0_ref, e1_ref, e2_ref):
    ea = ea_ref[...]
    outs = (e0_ref, e1_ref, e2_ref)
    for l in range(NLAYERS):
        halves = []
        for w_ref, b_ref in ((wea_ref, bea_ref), (web_ref, beb_ref)):
            wl = w_ref[l]
            wp = jnp.dot(wee_ref[...], wl, preferred_element_type=jnp.float32)
            bp = (jnp.dot(bee_ref[...], wl, preferred_element_type=jnp.float32)
                  + b_ref[l][None, :])
            halves.append(
                jnp.dot(ea, wp, preferred_element_type=jnp.float32) + bp)
        outs[l][...] = jnp.bitwise_or(
            _round_bf16_bits(halves[0]),
            jax.lax.shift_left(_round_bf16_bits(halves[1]), 16))


def _edge_e(edge_attr, w_ee, we_a, we_b, b_ee, be_a, be_b):
    d_edge = edge_attr.shape[1]
    espec = pl.BlockSpec((EB, H // 2), lambda i: (i, 0))
    wspec = pl.BlockSpec((NLAYERS, H, H // 2), lambda i: (0, 0, 0))
    bspec = pl.BlockSpec((NLAYERS, H // 2), lambda i: (0, 0))
    return pl.pallas_call(
        _edge_e_body,
        grid=(E // EB,),
        in_specs=[
            pl.BlockSpec((EB, d_edge), lambda i: (i, 0)),
            pl.BlockSpec((d_edge, H), lambda i: (0, 0)),
            wspec, wspec,
            pl.BlockSpec((1, H), lambda i: (0, 0)),
            bspec, bspec,
        ],
        out_specs=[espec, espec, espec],
        out_shape=[jax.ShapeDtypeStruct((E, H // 2), jnp.int32)] * NLAYERS,
    )(edge_attr, w_ee, we_a, we_b, b_ee, be_a, be_b)


@functools.cache
def _get_sc_msgpass():
    mesh = plsc.VectorSubcoreMesh(core_axis_name="c", subcore_axis_name="s",
                                  num_cores=1)
    return functools.partial(
        pl.kernel,
        out_type=jax.ShapeDtypeStruct((16, RPT, H), jnp.float32),
        mesh=mesh,
        compiler_params=pltpu.CompilerParams(needs_layout_passes=False),
        scratch_types=[
            pltpu.VMEM((GCH, CHUNK), jnp.int32),
            pltpu.VMEM((GCH, CHUNK), jnp.int32),
            pltpu.VMEM((CHUNK, H), jnp.float32),
            pltpu.VMEM((CHUNK, H), jnp.float32),
            pltpu.VMEM((CHUNK, H), jnp.float32),
            pltpu.VMEM((CHUNK, H), jnp.float32),
            pltpu.VMEM((CHUNK, H // 2), jnp.int32),
            pltpu.VMEM((CHUNK, H // 2), jnp.int32),
            pltpu.VMEM((CHUNK, H // 2), jnp.int32),
            pltpu.VMEM((CHUNK, H // 2), jnp.int32),
            pltpu.VMEM_SHARED((N, H), jnp.float32),
        ] + [pltpu.SemaphoreType.DMA] * 12,
    )(_sc_msgpass_body)


def _sc_msgpass_body(h_hbm, src_hbm, dst_hbm, e_hbm, out_hbm, srcs_v, dsts_v,
                     mbuf0, mbuf1, mbuf2, mbuf3, ebuf0, ebuf1, ebuf2, ebuf3,
                     aggr_sh, es0, es1, es2, es3, gs0, gs1, gs2, gs3,
                     ss0, ss1, ss2, ss3):
    sid = lax.axis_index("s")
    wid = sid
    mbufs = (mbuf0, mbuf1, mbuf2, mbuf3)
    ebufs = (ebuf0, ebuf1, ebuf2, ebuf3)
    esems = (es0, es1, es2, es3)
    gsems = (gs0, gs1, gs2, gs3)
    ssems = (ss0, ss1, ss2, ss3)

    def _zrow(r, carry):
        for j in range(H // 16):
            mbuf0[r, pl.ds(j * 16, 16)] = jnp.zeros((16,), jnp.float32)
        return carry
    lax.fori_loop(0, CHUNK, _zrow, 0)

    def _zcopy(k, carry):
        pltpu.sync_copy(mbuf0,
                        aggr_sh.at[pl.ds(sid * RPT + k * CHUNK, CHUNK), :])
        return carry
    lax.fori_loop(0, RPT // CHUNK, _zcopy, 0)
    _ztail = RPT - (RPT // CHUNK) * CHUNK
    pltpu.sync_copy(
        mbuf0.at[pl.ds(0, _ztail), :],
        aggr_sh.at[pl.ds(sid * RPT + (RPT // CHUNK) * CHUNK, _ztail), :])
    plsc.subcore_barrier()

    ebase = wid * EPW
    _MASK = jnp.full((16,), -65536, jnp.int32)
    _SH16 = jnp.full((16,), 16, jnp.int32)

    def _compute(b):
        def _rrow(r, rc):
            for j in range(H // 32):
                iv = ebufs[b][r, pl.ds(j * 16, 16)]
                lo = plsc.bitcast(jax.lax.shift_left(iv, _SH16), jnp.float32)
                hi = plsc.bitcast(jnp.bitwise_and(iv, _MASK), jnp.float32)
                sl0 = pl.ds(j * 32, 16)
                sl1 = pl.ds(j * 32 + 16, 16)
                mbufs[b][r, sl0] = jnp.maximum(mbufs[b][r, sl0] + lo, 0.0)
                mbufs[b][r, sl1] = jnp.maximum(mbufs[b][r, sl1] + hi, 0.0)
            return rc
        lax.fori_loop(0, CHUNK, _rrow, 0)

    def _group(gg, gcarry):
        pltpu.sync_copy(src_hbm.at[wid, gg], srcs_v)
        pltpu.sync_copy(dst_hbm.at[wid, gg], dsts_v)
        gbase = ebase + gg * (GCH * CHUNK)

        def e_start(b, l):
            pltpu.async_copy(e_hbm.at[pl.ds(gbase + l * CHUNK, CHUNK), :],
                             ebufs[b], esems[b])

        def e_wait(b, l):
            pltpu.make_async_copy(
                e_hbm.at[pl.ds(gbase + l * CHUNK, CHUNK), :],
                ebufs[b], esems[b]).wait()

        def g_start(b, l):
            pltpu.async_copy(h_hbm.at[srcs_v.at[l]], mbufs[b], gsems[b])

        def g_wait(b, l):
            pltpu.make_async_copy(h_hbm.at[srcs_v.at[l]], mbufs[b],
                                  gsems[b]).wait()

        def s_start(b, l):
            pltpu.async_copy(mbufs[b], aggr_sh.at[dsts_v.at[l]], ssems[b],
                             add=True)

        def s_wait(b, l):
            pltpu.make_async_copy(mbufs[b], aggr_sh.at[dsts_v.at[l]],
                                  ssems[b]).wait()

        e_start(0, 0)
        g_start(0, 0)
        e_start(1, 1)
        g_start(1, 1)

        def _quad(g, qcarry):
            l0 = g * 4
            for b in range(4):
                l = l0 + b
                b2 = (b + 2) % 4

                def _prefetch(lv=l + 2, bv=b2):
                    e_start(bv, lv)
                    g_start(bv, lv)

                if b < 2:
                    @pl.when(g >= 1)
                    def _(lv=l - 2, bv=b2):
                        s_wait(bv, lv)
                    _prefetch()
                else:
                    s_wait(b2, l - 2)
                    @pl.when(g < QUADS - 1)
                    def _(fn=_prefetch):
                        fn()
                e_wait(b, l)
                g_wait(b, l)
                _compute(b)
                s_start(b, l)
            return qcarry

        lax.fori_loop(0, QUADS, _quad, 0)
        s_wait(2, GCH - 2)
        s_wait(3, GCH - 1)
        return gcarry

    lax.fori_loop(0, NGROUPS, _group, 0)
    plsc.subcore_barrier()

    pltpu.sync_copy(aggr_sh.at[pl.ds(sid * RPT, RPT), :], out_hbm.at[sid])


def _layer_body(h_ref, p_ref, eps_ref, wm1_ref, bm1_ref, wm2_ref, bm2_ref,
                g_ref, b_ref, o_ref):
    h = h_ref[...]
    out = (1.0 + eps_ref[0, 0]) * h + p_ref[...]
    t = jnp.maximum(
        jnp.dot(out, wm1_ref[...], preferred_element_type=jnp.float32)
        + bm1_ref[...], 0.0)
    out = (jnp.dot(t, wm2_ref[...], preferred_element_type=jnp.float32)
           + bm2_ref[...])
    out = out * (g_ref[...] * _BN_SCALE) + b_ref[...] + h
    o_ref[...] = jnp.maximum(out, 0.0)


def _layer_tc(h, parts, eps_l, wm1, bm1, wm2, bm2, gamma_l, beta_l):
    return pl.pallas_call(
        _layer_body,
        grid=(N // NB,),
        in_specs=[
            pl.BlockSpec((NB, H), lambda i: (i, 0)),
            pl.BlockSpec((NB, H), lambda i: (i, 0)),
            pl.BlockSpec(memory_space=pltpu.SMEM),
            pl.BlockSpec((H, 2 * H), lambda i: (0, 0)),
            pl.BlockSpec((1, 2 * H), lambda i: (0, 0)),
            pl.BlockSpec((2 * H, H), lambda i: (0, 0)),
            pl.BlockSpec((1, H), lambda i: (0, 0)),
            pl.BlockSpec((1, H), lambda i: (0, 0)),
            pl.BlockSpec((1, H), lambda i: (0, 0)),
        ],
        out_specs=pl.BlockSpec((NB, H), lambda i: (i, 0)),
        out_shape=jax.ShapeDtypeStruct((N, H), jnp.float32),
    )(h, parts, eps_l, wm1, bm1, wm2, bm2, gamma_l, beta_l)


def _pool_body(h_ref, wo1_ref, bo1_ref, wo2_ref, bo2_ref, logits_ref,
               pooled_ref, acc_ref):
    i = pl.program_id(0)

    @pl.when(i == 0)
    def _():
        acc_ref[...] = jnp.zeros_like(acc_ref)

    acc_ref[...] += jnp.sum(h_ref[...], axis=0, keepdims=True)

    @pl.when(i == pl.num_programs(0) - 1)
    def _():
        pooled = acc_ref[...] * (1.0 / N)
        pooled_ref[...] = pooled
        t = jnp.maximum(
            jnp.dot(pooled, wo1_ref[...], preferred_element_type=jnp.float32)
            + bo1_ref[...], 0.0)
        logits_ref[...] = (
            jnp.dot(t, wo2_ref[...], preferred_element_type=jnp.float32)
            + bo2_ref[...])


def _pool_tc(h, wo1, bo1, wo2, bo2):
    h2 = wo1.shape[1]
    nout = wo2.shape[1]
    return pl.pallas_call(
        _pool_body,
        grid=(N // NB,),
        in_specs=[
            pl.BlockSpec((NB, H), lambda i: (i, 0)),
            pl.BlockSpec((H, h2), lambda i: (0, 0)),
            pl.BlockSpec((1, h2), lambda i: (0, 0)),
            pl.BlockSpec((h2, nout), lambda i: (0, 0)),
            pl.BlockSpec((1, nout), lambda i: (0, 0)),
        ],
        out_specs=[
            pl.BlockSpec((1, nout), lambda i: (0, 0)),
            pl.BlockSpec((1, H), lambda i: (0, 0)),
        ],
        out_shape=[
            jax.ShapeDtypeStruct((1, nout), jnp.float32),
            jax.ShapeDtypeStruct((1, H), jnp.float32),
        ],
        scratch_shapes=[pltpu.VMEM((1, H), jnp.float32)],
    )(h, wo1, bo1, wo2, bo2)


def kernel(x, edge_index, edge_attr, W_ne, b_ne, W_ee, b_ee, eps, We, be,
           Wm1, bm1, Wm2, bm2, gamma, beta, Wo1, bo1, Wo2, bo2):
    ei = edge_index.astype(jnp.int32)
    src_r = ei[0].reshape(NW, NGROUPS, GCH, CHUNK)
    dst_r = ei[1].reshape(NW, NGROUPS, GCH, CHUNK)

    h = _node_enc(x, W_ne, b_ne.reshape(1, H))
    sa = jnp.array(_SEL_A, dtype=jnp.int32)
    sb = jnp.array(_SEL_B, dtype=jnp.int32)
    e_all = _edge_e(edge_attr, W_ee, We[:, :, sa], We[:, :, sb],
                    b_ee.reshape(1, H), be[:, sa], be[:, sb])

    sc_msgpass = _get_sc_msgpass()
    for l in range(NLAYERS):
        parts = sc_msgpass(h, src_r, dst_r, e_all[l]).reshape(N, H)
        h = _layer_tc(h, parts, eps[l].reshape(1, 1), Wm1[l],
                      bm1[l].reshape(1, 2 * H), Wm2[l], bm2[l].reshape(1, H),
                      gamma[l].reshape(1, H), beta[l].reshape(1, H))

    return _pool_tc(h, Wo1, bo1.reshape(1, H // 2), Wo2,
                    bo2.reshape(1, Wo2.shape[1]))
